# baseline (speedup 1.0000x reference)
import functools

import jax
import jax.numpy as jnp
import numpy as _np
from jax import lax
from jax.experimental import pallas as pl
from jax.experimental.pallas import tpu as pltpu
from jax.experimental.pallas import tpu_sc as plsc

N = 10000
E = 320000
F_IN = 128
HID = 128
NCLASS = 16

NC = 2
NS = 16
CH = 128
NB = 4
RPT = N // NS
E2 = E // 2
ES = 4000
NCAP = 83
CAP = NCAP * CH

_SC_PARAMS = pltpu.CompilerParams(
    use_tc_tiling_on_sc=False, needs_layout_passes=False)

_PERM64 = _np.zeros(64, _np.int32)
for _j in (0, 1):
  for _i in range(16):
    _PERM64[32 * _j + _i] = 32 * _j + 2 * _i
    _PERM64[32 * _j + 16 + _i] = 32 * _j + 2 * _i + 1
_PERM128 = _np.concatenate([_PERM64, 64 + _PERM64])


_mesh = plsc.VectorSubcoreMesh(core_axis_name="c", subcore_axis_name="s")


@functools.partial(
    pl.kernel,
    out_type=[
        jax.ShapeDtypeStruct((NS, NC, CAP), jnp.int32),
        jax.ShapeDtypeStruct((NS, NC, CAP), jnp.int32),
        jax.ShapeDtypeStruct((NS, NC, CAP), jnp.float32),
    ],
    mesh=_mesh,
    compiler_params=_SC_PARAMS,
    scratch_types=[
        [pltpu.VMEM((ES,), jnp.int32) for _ in range(2)],
        [pltpu.VMEM((ES,), jnp.int32) for _ in range(2)],
        [pltpu.VMEM((ES,), jnp.float32) for _ in range(2)],
        pltpu.VMEM((CAP,), jnp.int32),
        pltpu.VMEM((CAP,), jnp.int32),
        pltpu.VMEM((CAP,), jnp.float32),
        [pltpu.SemaphoreType.DMA for _ in range(2)],
    ],
)
def _bucket(src_hbm, dst_hbm, w_hbm, srcb_hbm, dstb_hbm, wb_hbm,
            src_st, dst_st, w_st, src_l, dst_l, w_l, sems):
  half = lax.axis_index("c")
  r = lax.axis_index("s")
  lo = r * RPT

  def zfill(i, carry):
    src_l[pl.ds(16 * i, 16)] = jnp.zeros((16,), jnp.int32)
    dst_l[pl.ds(16 * i, 16)] = jnp.zeros((16,), jnp.int32)
    w_l[pl.ds(16 * i, 16)] = jnp.zeros((16,), jnp.float32)
    return carry
  lax.fori_loop(0, CAP // 16, zfill, 0)

  def stage(step, b):
    base = step * ES
    pltpu.async_copy(src_hbm.at[half, pl.ds(base, ES)], src_st[b], sems[b])
    pltpu.async_copy(dst_hbm.at[half, pl.ds(base, ES)], dst_st[b], sems[b])
    pltpu.async_copy(w_hbm.at[half, pl.ds(base, ES)], w_st[b], sems[b])

  def wait_stage(b):
    pltpu.make_async_copy(src_hbm.at[0, pl.ds(0, ES)], src_st[b],
                          sems[b]).wait()
    pltpu.make_async_copy(dst_hbm.at[0, pl.ds(0, ES)], dst_st[b],
                          sems[b]).wait()
    pltpu.make_async_copy(w_hbm.at[0, pl.ds(0, ES)], w_st[b], sems[b]).wait()

  stage(0, 0)
  nsteps = E2 // ES

  def phase(step, b, off):
    @pl.when(step + 1 < nsteps)
    def _():
      stage(step + 1, 1 - b)
    wait_stage(b)

    def vec_body(g, off2):
      dv = dst_st[b][pl.ds(16 * g, 16)]
      m = (dv >= lo) & (dv < lo + RPT)
      off2 = jnp.minimum(off2, CAP - 16)
      plsc.store_compressed(src_l.at[pl.ds(off2, 16)],
                            src_st[b][pl.ds(16 * g, 16)], mask=m)
      plsc.store_compressed(dst_l.at[pl.ds(off2, 16)], dv - lo, mask=m)
      plsc.store_compressed(w_l.at[pl.ds(off2, 16)],
                            w_st[b][pl.ds(16 * g, 16)], mask=m)
      return off2 + jnp.sum(m.astype(jnp.int32))
    return lax.fori_loop(0, ES // 16, vec_body, off)

  def round_body(r2, off):
    off = phase(2 * r2, 0, off)
    return phase(2 * r2 + 1, 1, off)
  lax.fori_loop(0, nsteps // 2, round_body, jnp.int32(0))

  pltpu.sync_copy(src_l, srcb_hbm.at[r].at[half])
  pltpu.sync_copy(dst_l, dstb_hbm.at[r].at[half])
  pltpu.sync_copy(w_l, wb_hbm.at[r].at[half])


def _splat(wvec, k):
  return wvec.at[jnp.full((16,), k, jnp.int32)].get(mode="promise_in_bounds")


def _make_spmm_fsplit():
  nch2 = NC * NCAP

  @functools.partial(
      pl.kernel,
      out_type=jax.ShapeDtypeStruct((NC, N, 64), jnp.float32),
      mesh=_mesh,
      compiler_params=_SC_PARAMS,
      scratch_types=[
          pltpu.VMEM((nch2, CH), jnp.int32),
          pltpu.VMEM((NC * CAP,), jnp.float32),
          pltpu.VMEM((nch2, CH), jnp.int32),
          [pltpu.VMEM((CH, 64), jnp.bfloat16) for _ in range(NB)],
          pltpu.VMEM((RPT, 64), jnp.float32),
          [pltpu.SemaphoreType.DMA for _ in range(NB)],
      ],
  )
  def spmm(srcb_hbm, dstb_hbm, wb_hbm, x0_hbm, x1_hbm, out_hbm,
           src_v, w_v, dst_v, rows, acc, gsems):
    cid = lax.axis_index("c")
    sid = lax.axis_index("s")

    pltpu.sync_copy(srcb_hbm.at[sid], src_v)
    pltpu.sync_copy(wb_hbm.at[sid], w_v)
    pltpu.sync_copy(dstb_hbm.at[sid], dst_v)

    def zrow(i, carry):
      for j in range(4):
        acc[i, pl.ds(16 * j, 16)] = jnp.zeros((16,), jnp.float32)
      return carry
    lax.fori_loop(0, RPT, zrow, 0)

    for xhalf, x_hbm in ((0, x0_hbm), (1, x1_hbm)):
      @pl.when(cid == xhalf)
      def _():
        for b in range(NB - 1):
          pltpu.async_copy(x_hbm.at[src_v.at[b]], rows[b], gsems[b])

        def phase(c, k, prefetch):
          pltpu.make_async_copy(x_hbm.at[src_v.at[0]], rows[k],
                                gsems[k]).wait()
          if prefetch:
            cp = c + NB - 1

            @pl.when(cp < nch2)
            def _():
              pltpu.async_copy(x_hbm.at[src_v.at[cp]], rows[(k + NB - 1) % NB],
                               gsems[(k + NB - 1) % NB])

          def group(g, carry2):
            wvec = w_v[pl.ds(c * CH + 16 * g, 16)]
            dvec = dst_v[c, pl.ds(16 * g, 16)]
            lanes = lax.iota(jnp.int32, 16)
            for k2 in range(16):
              e = 16 * g + k2
              wsp = _splat(wvec, k2)
              dl = jnp.sum(dvec * (lanes == k2).astype(jnp.int32))
              for j in range(2):
                vb = rows[k][e, pl.ds(32 * j, 32)]
                va, vc = plsc.unpack(vb, format=plsc.PackFormat.INTERLEAVED)
                sa = pl.ds(32 * j, 16)
                sc = pl.ds(32 * j + 16, 16)
                acc[dl, sa] = acc[dl, sa] + va * wsp
                acc[dl, sc] = acc[dl, sc] + vc * wsp
            return carry2
          lax.fori_loop(0, CH // 16, group, 0)

        rounds = nch2 // NB

        def round_body(r2, carry):
          for k in range(NB):
            phase(r2 * NB + k, k, True)
          return carry
        lax.fori_loop(0, rounds, round_body, 0)
        for c in range(rounds * NB, nch2):
          phase(c, c % NB, False)

    pltpu.sync_copy(acc, out_hbm.at[cid].at[pl.ds(sid * RPT, RPT)])

  return spmm


def _make_spmm_e16():
  @functools.partial(
      pl.kernel,
      out_type=jax.ShapeDtypeStruct((NC, N, NCLASS), jnp.float32),
      mesh=_mesh,
      compiler_params=_SC_PARAMS,
      scratch_types=[
          pltpu.VMEM((NCAP, CH), jnp.int32),
          pltpu.VMEM((CAP,), jnp.float32),
          pltpu.VMEM((NCAP, CH), jnp.int32),
          [pltpu.VMEM((CH, NCLASS), jnp.float32) for _ in range(NB)],
          pltpu.VMEM((RPT, NCLASS), jnp.float32),
          [pltpu.SemaphoreType.DMA for _ in range(NB)],
      ],
  )
  def spmm(srcb_hbm, dstb_hbm, wb_hbm, t_hbm, out_hbm,
           src_v, w_v, dst_v, rows, acc, gsems):
    cid = lax.axis_index("c")
    sid = lax.axis_index("s")

    pltpu.sync_copy(srcb_hbm.at[sid].at[cid], src_v)
    pltpu.sync_copy(wb_hbm.at[sid].at[cid], w_v)
    pltpu.sync_copy(dstb_hbm.at[sid].at[cid], dst_v)

    def zrow(i, carry):
      acc[i, pl.ds(0, 16)] = jnp.zeros((16,), jnp.float32)
      return carry
    lax.fori_loop(0, RPT, zrow, 0)

    for b in range(NB - 1):
      pltpu.async_copy(t_hbm.at[src_v.at[b]], rows[b], gsems[b])

    def phase(c, k, prefetch):
      pltpu.make_async_copy(t_hbm.at[src_v.at[0]], rows[k], gsems[k]).wait()
      if prefetch:
        cp = c + NB - 1

        @pl.when(cp < NCAP)
        def _():
          pltpu.async_copy(t_hbm.at[src_v.at[cp]], rows[(k + NB - 1) % NB],
                           gsems[(k + NB - 1) % NB])

      def group(g, carry2):
        wvec = w_v[pl.ds(c * CH + 16 * g, 16)]
        dvec = dst_v[c, pl.ds(16 * g, 16)]
        lanes = lax.iota(jnp.int32, 16)
        for k2 in range(16):
          e = 16 * g + k2
          wsp = _splat(wvec, k2)
          dl = jnp.sum(dvec * (lanes == k2).astype(jnp.int32))
          acc[dl, pl.ds(0, 16)] = (acc[dl, pl.ds(0, 16)]
                                   + rows[k][e, pl.ds(0, 16)] * wsp)
        return carry2
      lax.fori_loop(0, CH // 16, group, 0)

    rounds = NCAP // NB

    def round_body(r2, carry):
      for k in range(NB):
        phase(r2 * NB + k, k, True)
      return carry
    lax.fori_loop(0, rounds, round_body, 0)
    for c in range(rounds * NB, NCAP):
      phase(c, c % NB, False)

    pltpu.sync_copy(acc, out_hbm.at[cid].at[pl.ds(sid * RPT, RPT)])

  return spmm


_spmm_f64 = _make_spmm_fsplit()
_spmm_e16 = _make_spmm_e16()


_RB = 2000


def _dense_body(pl_ref, ph_ref, w_ref, b_ref, o0_ref, o1_ref):
  h = jnp.concatenate([pl_ref[0], ph_ref[0]], axis=1)
  y = jnp.dot(h, w_ref[...], preferred_element_type=jnp.float32) + b_ref[...]
  y = jnp.maximum(y, 0.0)
  o0_ref[...] = y[:, :64].astype(jnp.bfloat16)
  o1_ref[...] = y[:, 64:].astype(jnp.bfloat16)


def _dense_relu(p, w, b):
  return pl.pallas_call(
      _dense_body,
      grid=(N // _RB,),
      in_specs=[
          pl.BlockSpec((1, _RB, 64), lambda i: (0, i, 0)),
          pl.BlockSpec((1, _RB, 64), lambda i: (1, i, 0)),
          pl.BlockSpec((HID, HID), lambda i: (0, 0)),
          pl.BlockSpec((1, HID), lambda i: (0, 0)),
      ],
      out_specs=[
          pl.BlockSpec((_RB, 64), lambda i: (i, 0)),
          pl.BlockSpec((_RB, 64), lambda i: (i, 0)),
      ],
      out_shape=[
          jax.ShapeDtypeStruct((N, 64), jnp.bfloat16),
          jax.ShapeDtypeStruct((N, 64), jnp.bfloat16),
      ],
  )(p, p, w, b)


def _dense2_body(pl_ref, ph_ref, w2_ref, b2_ref, w3_ref, vis_ref, t_ref):
  h = jnp.concatenate([pl_ref[0], ph_ref[0]], axis=1)
  h2 = jnp.maximum(
      jnp.dot(h, w2_ref[...], preferred_element_type=jnp.float32)
      + b2_ref[...], 0.0)
  vis_ref[...] = h2
  t_ref[...] = jnp.dot(h2, w3_ref[...], preferred_element_type=jnp.float32)


def _dense2(p, w2, b2, w3):
  return pl.pallas_call(
      _dense2_body,
      grid=(N // _RB,),
      in_specs=[
          pl.BlockSpec((1, _RB, 64), lambda i: (0, i, 0)),
          pl.BlockSpec((1, _RB, 64), lambda i: (1, i, 0)),
          pl.BlockSpec((HID, HID), lambda i: (0, 0)),
          pl.BlockSpec((1, HID), lambda i: (0, 0)),
          pl.BlockSpec((HID, NCLASS), lambda i: (0, 0)),
      ],
      out_specs=[
          pl.BlockSpec((_RB, HID), lambda i: (i, 0)),
          pl.BlockSpec((_RB, NCLASS), lambda i: (i, 0)),
      ],
      out_shape=[
          jax.ShapeDtypeStruct((N, HID), jnp.float32),
          jax.ShapeDtypeStruct((N, NCLASS), jnp.float32),
      ],
  )(p, p, w2, b2, w3)


def _final_body(p0_ref, p1_ref, b_ref, o_ref):
  logits = p0_ref[0] + p1_ref[0] + b_ref[...]
  m = jnp.max(logits, axis=1, keepdims=True)
  s = logits - m
  lse = jnp.log(jnp.sum(jnp.exp(s), axis=1, keepdims=True))
  o_ref[...] = s - lse


def _final(p, b3):
  return pl.pallas_call(
      _final_body,
      grid=(N // _RB,),
      in_specs=[
          pl.BlockSpec((1, _RB, NCLASS), lambda i: (0, i, 0)),
          pl.BlockSpec((1, _RB, NCLASS), lambda i: (1, i, 0)),
          pl.BlockSpec((1, NCLASS), lambda i: (0, 0)),
      ],
      out_specs=pl.BlockSpec((_RB, NCLASS), lambda i: (i, 0)),
      out_shape=jax.ShapeDtypeStruct((N, NCLASS), jnp.float32),
  )(p, p, b3)


def kernel(x, edge_index, edge_weight, W1, b1, W2, b2, W3, b3):
  src2 = edge_index[1].astype(jnp.int32).reshape(NC, E2)
  dst2 = edge_index[0].astype(jnp.int32).reshape(NC, E2)
  ew2 = edge_weight.reshape(NC, E2)

  srcb, dstb, wb = _bucket(src2, dst2, ew2)
  srcb_f = srcb.reshape(NS, NC * NCAP, CH)
  dstb_f = dstb.reshape(NS, NC * NCAP, CH)
  wb_f = wb.reshape(NS, NC * CAP)
  srcb_e = srcb.reshape(NS, NC, NCAP, CH)
  dstb_e = dstb.reshape(NS, NC, NCAP, CH)

  xb = x.astype(jnp.bfloat16)
  p0 = _spmm_f64(srcb_f, dstb_f, wb_f, xb[:, :64], xb[:, 64:])
  h1b0, h1b1 = _dense_relu(p0, W1[_PERM128], b1.reshape(1, HID))
  p1 = _spmm_f64(srcb_f, dstb_f, wb_f, h1b0, h1b1)
  vis, t2 = _dense2(p1, W2[_PERM128], b2.reshape(1, HID), W3)
  p2 = _spmm_e16(srcb_e, dstb_e, wb, t2)
  logp = _final(p2, b3.reshape(1, NCLASS))
  return (logp, vis)


# speedup vs baseline: 1.1081x; 1.1081x over previous
import functools

import jax
import jax.numpy as jnp
import numpy as _np
from jax import lax
from jax.experimental import pallas as pl
from jax.experimental.pallas import tpu as pltpu
from jax.experimental.pallas import tpu_sc as plsc

N = 10000
E = 320000
F_IN = 128
HID = 128
NCLASS = 16

NC = 2
NS = 16
CH = 128
NB = 4
RPT = N // NS
E2 = E // 2
ES = 4000
NCAP = 83
CAP = NCAP * CH

_SC_PARAMS = pltpu.CompilerParams(
    use_tc_tiling_on_sc=False, needs_layout_passes=False)

_PERM64 = _np.zeros(64, _np.int32)
for _j in (0, 1):
  for _i in range(16):
    _PERM64[32 * _j + _i] = 32 * _j + 2 * _i
    _PERM64[32 * _j + 16 + _i] = 32 * _j + 2 * _i + 1
_PERM128 = _np.concatenate([_PERM64, 64 + _PERM64])


_mesh = plsc.VectorSubcoreMesh(core_axis_name="c", subcore_axis_name="s")


@functools.partial(
    pl.kernel,
    out_type=[
        jax.ShapeDtypeStruct((NS, NC, CAP), jnp.int32),
        jax.ShapeDtypeStruct((NS, NC, CAP), jnp.int32),
        jax.ShapeDtypeStruct((NS, NC, CAP), jnp.float32),
    ],
    mesh=_mesh,
    compiler_params=_SC_PARAMS,
    scratch_types=[
        [pltpu.VMEM((ES,), jnp.int32) for _ in range(2)],
        [pltpu.VMEM((ES,), jnp.int32) for _ in range(2)],
        [pltpu.VMEM((ES,), jnp.float32) for _ in range(2)],
        pltpu.VMEM((CAP,), jnp.int32),
        pltpu.VMEM((CAP,), jnp.int32),
        pltpu.VMEM((CAP,), jnp.float32),
        [pltpu.SemaphoreType.DMA for _ in range(2)],
    ],
)
def _bucket(src_hbm, dst_hbm, w_hbm, srcb_hbm, dstb_hbm, wb_hbm,
            src_st, dst_st, w_st, src_l, dst_l, w_l, sems):
  half = lax.axis_index("c")
  r = lax.axis_index("s")
  lo = r * RPT

  def zfill(i, carry):
    src_l[pl.ds(16 * i, 16)] = jnp.zeros((16,), jnp.int32)
    dst_l[pl.ds(16 * i, 16)] = jnp.zeros((16,), jnp.int32)
    w_l[pl.ds(16 * i, 16)] = jnp.zeros((16,), jnp.float32)
    return carry
  lax.fori_loop(0, CAP // 16, zfill, 0)

  def stage(step, b):
    base = step * ES
    pltpu.async_copy(src_hbm.at[half, pl.ds(base, ES)], src_st[b], sems[b])
    pltpu.async_copy(dst_hbm.at[half, pl.ds(base, ES)], dst_st[b], sems[b])
    pltpu.async_copy(w_hbm.at[half, pl.ds(base, ES)], w_st[b], sems[b])

  def wait_stage(b):
    pltpu.make_async_copy(src_hbm.at[0, pl.ds(0, ES)], src_st[b],
                          sems[b]).wait()
    pltpu.make_async_copy(dst_hbm.at[0, pl.ds(0, ES)], dst_st[b],
                          sems[b]).wait()
    pltpu.make_async_copy(w_hbm.at[0, pl.ds(0, ES)], w_st[b], sems[b]).wait()

  stage(0, 0)
  nsteps = E2 // ES

  def phase(step, b, off):
    @pl.when(step + 1 < nsteps)
    def _():
      stage(step + 1, 1 - b)
    wait_stage(b)

    def vec_body(g, off2):
      dv = dst_st[b][pl.ds(16 * g, 16)]
      m = (dv >= lo) & (dv < lo + RPT)
      off2 = jnp.minimum(off2, CAP - 16)
      plsc.store_compressed(src_l.at[pl.ds(off2, 16)],
                            src_st[b][pl.ds(16 * g, 16)], mask=m)
      plsc.store_compressed(dst_l.at[pl.ds(off2, 16)], dv - lo, mask=m)
      plsc.store_compressed(w_l.at[pl.ds(off2, 16)],
                            w_st[b][pl.ds(16 * g, 16)], mask=m)
      return off2 + jnp.sum(m.astype(jnp.int32))
    return lax.fori_loop(0, ES // 16, vec_body, off)

  def round_body(r2, off):
    off = phase(2 * r2, 0, off)
    return phase(2 * r2 + 1, 1, off)
  lax.fori_loop(0, nsteps // 2, round_body, jnp.int32(0))

  pltpu.sync_copy(src_l, srcb_hbm.at[r].at[half])
  pltpu.sync_copy(dst_l, dstb_hbm.at[r].at[half])
  pltpu.sync_copy(w_l, wb_hbm.at[r].at[half])


def _splat(wvec, k):
  return wvec.at[jnp.full((16,), k, jnp.int32)].get(mode="promise_in_bounds")


def _make_spmm_fsplit():
  nch2 = NC * NCAP

  @functools.partial(
      pl.kernel,
      out_type=jax.ShapeDtypeStruct((NC, N, 64), jnp.float32),
      mesh=_mesh,
      compiler_params=_SC_PARAMS,
      scratch_types=[
          pltpu.VMEM((nch2, CH), jnp.int32),
          pltpu.VMEM((NC * CAP,), jnp.float32),
          pltpu.VMEM((nch2, CH), jnp.int32),
          [pltpu.VMEM((CH, 64), jnp.bfloat16) for _ in range(NB)],
          pltpu.VMEM((RPT, 64), jnp.float32),
          [pltpu.SemaphoreType.DMA for _ in range(NB)],
      ],
  )
  def spmm(srcb_hbm, dstb_hbm, wb_hbm, x0_hbm, x1_hbm, out_hbm,
           src_v, w_v, dst_v, rows, acc, gsems):
    cid = lax.axis_index("c")
    sid = lax.axis_index("s")

    pltpu.sync_copy(srcb_hbm.at[sid], src_v)
    pltpu.sync_copy(wb_hbm.at[sid], w_v)
    pltpu.sync_copy(dstb_hbm.at[sid], dst_v)

    def zrow(i, carry):
      for j in range(4):
        acc[i, pl.ds(16 * j, 16)] = jnp.zeros((16,), jnp.float32)
      return carry
    lax.fori_loop(0, RPT, zrow, 0)

    for xhalf, x_hbm in ((0, x0_hbm), (1, x1_hbm)):
      @pl.when(cid == xhalf)
      def _():
        for b in range(NB - 1):
          pltpu.async_copy(x_hbm.at[src_v.at[b]], rows[b], gsems[b])

        def phase(c, k, prefetch):
          pltpu.make_async_copy(x_hbm.at[src_v.at[0]], rows[k],
                                gsems[k]).wait()
          if prefetch:
            cp = c + NB - 1

            @pl.when(cp < nch2)
            def _():
              pltpu.async_copy(x_hbm.at[src_v.at[cp]], rows[(k + NB - 1) % NB],
                               gsems[(k + NB - 1) % NB])

          lanes = lax.iota(jnp.int32, 16)

          def group(g, carry2):
            wvec = w_v[pl.ds(c * CH + 16 * g, 16)]
            dvec = dst_v[c, pl.ds(16 * g, 16)]

            def edge(k2, carry3):
              e = 16 * g + k2
              wsp = _splat(wvec, k2)
              dls = _splat(dvec, k2)
              for j in range(2):
                vb = rows[k][e, pl.ds(32 * j, 32)]
                va, vc = plsc.unpack(vb, format=plsc.PackFormat.INTERLEAVED)
                plsc.addupdate_scatter(acc, [dls, lanes + 32 * j], va * wsp)
                plsc.addupdate_scatter(acc, [dls, lanes + 32 * j + 16],
                                       vc * wsp)
              return carry3
            lax.fori_loop(0, 16, edge, 0, unroll=4)
            return carry2
          lax.fori_loop(0, CH // 16, group, 0)

        rounds = nch2 // NB

        def round_body(r2, carry):
          for k in range(NB):
            phase(r2 * NB + k, k, True)
          return carry
        lax.fori_loop(0, rounds, round_body, 0)
        for c in range(rounds * NB, nch2):
          phase(c, c % NB, False)

    pltpu.sync_copy(acc, out_hbm.at[cid].at[pl.ds(sid * RPT, RPT)])

  return spmm


def _make_spmm_e16():
  @functools.partial(
      pl.kernel,
      out_type=jax.ShapeDtypeStruct((NC, N, NCLASS), jnp.float32),
      mesh=_mesh,
      compiler_params=_SC_PARAMS,
      scratch_types=[
          pltpu.VMEM((NCAP, CH), jnp.int32),
          pltpu.VMEM((CAP,), jnp.float32),
          pltpu.VMEM((NCAP, CH), jnp.int32),
          [pltpu.VMEM((CH, NCLASS), jnp.float32) for _ in range(NB)],
          pltpu.VMEM((RPT, NCLASS), jnp.float32),
          [pltpu.SemaphoreType.DMA for _ in range(NB)],
      ],
  )
  def spmm(srcb_hbm, dstb_hbm, wb_hbm, t_hbm, out_hbm,
           src_v, w_v, dst_v, rows, acc, gsems):
    cid = lax.axis_index("c")
    sid = lax.axis_index("s")

    pltpu.sync_copy(srcb_hbm.at[sid].at[cid], src_v)
    pltpu.sync_copy(wb_hbm.at[sid].at[cid], w_v)
    pltpu.sync_copy(dstb_hbm.at[sid].at[cid], dst_v)

    def zrow(i, carry):
      acc[i, pl.ds(0, 16)] = jnp.zeros((16,), jnp.float32)
      return carry
    lax.fori_loop(0, RPT, zrow, 0)

    for b in range(NB - 1):
      pltpu.async_copy(t_hbm.at[src_v.at[b]], rows[b], gsems[b])

    def phase(c, k, prefetch):
      pltpu.make_async_copy(t_hbm.at[src_v.at[0]], rows[k], gsems[k]).wait()
      if prefetch:
        cp = c + NB - 1

        @pl.when(cp < NCAP)
        def _():
          pltpu.async_copy(t_hbm.at[src_v.at[cp]], rows[(k + NB - 1) % NB],
                           gsems[(k + NB - 1) % NB])

      lanes = lax.iota(jnp.int32, 16)

      def group(g, carry2):
        wvec = w_v[pl.ds(c * CH + 16 * g, 16)]
        dvec = dst_v[c, pl.ds(16 * g, 16)]

        def edge(k2, carry3):
          e = 16 * g + k2
          wsp = _splat(wvec, k2)
          dls = _splat(dvec, k2)
          plsc.addupdate_scatter(acc, [dls, lanes],
                                 rows[k][e, pl.ds(0, 16)] * wsp)
          return carry3
        lax.fori_loop(0, 16, edge, 0, unroll=8)
        return carry2
      lax.fori_loop(0, CH // 16, group, 0)

    rounds = NCAP // NB

    def round_body(r2, carry):
      for k in range(NB):
        phase(r2 * NB + k, k, True)
      return carry
    lax.fori_loop(0, rounds, round_body, 0)
    for c in range(rounds * NB, NCAP):
      phase(c, c % NB, False)

    pltpu.sync_copy(acc, out_hbm.at[cid].at[pl.ds(sid * RPT, RPT)])

  return spmm


_spmm_f64 = _make_spmm_fsplit()
_spmm_e16 = _make_spmm_e16()


_RB = 2000


def _dense_body(pl_ref, ph_ref, w_ref, b_ref, o0_ref, o1_ref):
  h = jnp.concatenate([pl_ref[0], ph_ref[0]], axis=1)
  y = jnp.dot(h, w_ref[...], preferred_element_type=jnp.float32) + b_ref[...]
  y = jnp.maximum(y, 0.0)
  o0_ref[...] = y[:, :64].astype(jnp.bfloat16)
  o1_ref[...] = y[:, 64:].astype(jnp.bfloat16)


def _dense_relu(p, w, b):
  return pl.pallas_call(
      _dense_body,
      grid=(N // _RB,),
      in_specs=[
          pl.BlockSpec((1, _RB, 64), lambda i: (0, i, 0)),
          pl.BlockSpec((1, _RB, 64), lambda i: (1, i, 0)),
          pl.BlockSpec((HID, HID), lambda i: (0, 0)),
          pl.BlockSpec((1, HID), lambda i: (0, 0)),
      ],
      out_specs=[
          pl.BlockSpec((_RB, 64), lambda i: (i, 0)),
          pl.BlockSpec((_RB, 64), lambda i: (i, 0)),
      ],
      out_shape=[
          jax.ShapeDtypeStruct((N, 64), jnp.bfloat16),
          jax.ShapeDtypeStruct((N, 64), jnp.bfloat16),
      ],
  )(p, p, w, b)


def _dense2_body(pl_ref, ph_ref, w2_ref, b2_ref, w3_ref, vis_ref, t_ref):
  h = jnp.concatenate([pl_ref[0], ph_ref[0]], axis=1)
  h2 = jnp.maximum(
      jnp.dot(h, w2_ref[...], preferred_element_type=jnp.float32)
      + b2_ref[...], 0.0)
  vis_ref[...] = h2
  t_ref[...] = jnp.dot(h2, w3_ref[...], preferred_element_type=jnp.float32)


def _dense2(p, w2, b2, w3):
  return pl.pallas_call(
      _dense2_body,
      grid=(N // _RB,),
      in_specs=[
          pl.BlockSpec((1, _RB, 64), lambda i: (0, i, 0)),
          pl.BlockSpec((1, _RB, 64), lambda i: (1, i, 0)),
          pl.BlockSpec((HID, HID), lambda i: (0, 0)),
          pl.BlockSpec((1, HID), lambda i: (0, 0)),
          pl.BlockSpec((HID, NCLASS), lambda i: (0, 0)),
      ],
      out_specs=[
          pl.BlockSpec((_RB, HID), lambda i: (i, 0)),
          pl.BlockSpec((_RB, NCLASS), lambda i: (i, 0)),
      ],
      out_shape=[
          jax.ShapeDtypeStruct((N, HID), jnp.float32),
          jax.ShapeDtypeStruct((N, NCLASS), jnp.float32),
      ],
  )(p, p, w2, b2, w3)


def _final_body(p0_ref, p1_ref, b_ref, o_ref):
  logits = p0_ref[0] + p1_ref[0] + b_ref[...]
  m = jnp.max(logits, axis=1, keepdims=True)
  s = logits - m
  lse = jnp.log(jnp.sum(jnp.exp(s), axis=1, keepdims=True))
  o_ref[...] = s - lse


def _final(p, b3):
  return pl.pallas_call(
      _final_body,
      grid=(N // _RB,),
      in_specs=[
          pl.BlockSpec((1, _RB, NCLASS), lambda i: (0, i, 0)),
          pl.BlockSpec((1, _RB, NCLASS), lambda i: (1, i, 0)),
          pl.BlockSpec((1, NCLASS), lambda i: (0, 0)),
      ],
      out_specs=pl.BlockSpec((_RB, NCLASS), lambda i: (i, 0)),
      out_shape=jax.ShapeDtypeStruct((N, NCLASS), jnp.float32),
  )(p, p, b3)


def kernel(x, edge_index, edge_weight, W1, b1, W2, b2, W3, b3):
  src2 = edge_index[1].astype(jnp.int32).reshape(NC, E2)
  dst2 = edge_index[0].astype(jnp.int32).reshape(NC, E2)
  ew2 = edge_weight.reshape(NC, E2)

  srcb, dstb, wb = _bucket(src2, dst2, ew2)
  srcb_f = srcb.reshape(NS, NC * NCAP, CH)
  dstb_f = dstb.reshape(NS, NC * NCAP, CH)
  wb_f = wb.reshape(NS, NC * CAP)
  srcb_e = srcb.reshape(NS, NC, NCAP, CH)
  dstb_e = dstb.reshape(NS, NC, NCAP, CH)

  xb = x.astype(jnp.bfloat16)
  p0 = _spmm_f64(srcb_f, dstb_f, wb_f, xb[:, :64], xb[:, 64:])
  h1b0, h1b1 = _dense_relu(p0, W1[_PERM128], b1.reshape(1, HID))
  p1 = _spmm_f64(srcb_f, dstb_f, wb_f, h1b0, h1b1)
  vis, t2 = _dense2(p1, W2[_PERM128], b2.reshape(1, HID), W3)
  p2 = _spmm_e16(srcb_e, dstb_e, wb, t2)
  logp = _final(p2, b3.reshape(1, NCLASS))
  return (logp, vis)


# speedup vs baseline: 2.1307x; 1.9228x over previous
import functools

import jax
import jax.numpy as jnp
from jax import lax
from jax.experimental import pallas as pl
from jax.experimental.pallas import tpu as pltpu
from jax.experimental.pallas import tpu_sc as plsc

N = 10000
E = 320000
F_IN = 128
HID = 128
NCLASS = 16

NC = 2
NS = 16
CH = 80
NB_F = 4
NB_E = 4
N_PAD = 10000
RPT = N_PAD // NS
ZR = 80

NCH_F = -(-E // (NS * CH))
EPT_F = NCH_F * CH
NCH_E = -(-E // (NC * NS * CH))
EPT_E = NCH_E * CH


import numpy as _np

_PERM64 = _np.zeros(64, _np.int32)
for _j in (0, 1):
  for _i in range(16):
    _PERM64[32 * _j + _i] = 32 * _j + 2 * _i
    _PERM64[32 * _j + 16 + _i] = 32 * _j + 2 * _i + 1
_PERM128 = _np.concatenate([_PERM64, 64 + _PERM64])


def _scale_convert(rows_b, scat_v, w_v, c):
  def group(g, carry):
    wvec = w_v[pl.ds(c * CH + 16 * g, 16)]
    for k in range(16):
      wsp = wvec.at[jnp.full((16,), k, jnp.int32)].get(
          mode="promise_in_bounds")
      e = 16 * g + k
      for j in range(2):
        vb = rows_b[e, pl.ds(32 * j, 32)]
        lo, hi = plsc.unpack(vb, format=plsc.PackFormat.INTERLEAVED)
        scat_v[e, pl.ds(32 * j, 16)] = lo * wsp
        scat_v[e, pl.ds(32 * j + 16, 16)] = hi * wsp
    return carry
  lax.fori_loop(0, CH // 16, group, 0)


def _run_edges_bf16(x_hbm, src_v, dst_v, w_v, rows_b, scat, gsems, ssems,
                    acc, nch):
  NB = len(rows_b)

  def wait_gather(b, c):
    pltpu.make_async_copy(x_hbm.at[src_v.at[c]], rows_b[b], gsems[b]).wait()

  def wait_scatter(b):
    pltpu.make_async_copy(scat[b], acc.at[dst_v.at[0]], ssems[b]).wait()

  def phase(c, b, prefetch, first_round):
    wait_gather(b, c)
    if not first_round:
      wait_scatter(b)
    _scale_convert(rows_b[b], scat[b], w_v, c)
    pltpu.async_copy(scat[b], acc.at[dst_v.at[c]], ssems[b], add=True)
    if prefetch:
      cp = c + (NB - 1)
      bp = (b + NB - 1) % NB

      @pl.when(cp < nch)
      def _():
        pltpu.async_copy(x_hbm.at[src_v.at[cp]], rows_b[bp], gsems[bp])

  for b in range(NB - 1):
    pltpu.async_copy(x_hbm.at[src_v.at[b]], rows_b[b], gsems[b])

  for k in range(NB):
    phase(k, k, True, True)

  rounds = nch // NB

  def round_body(r, carry):
    for k in range(NB):
      phase(r * NB + k, k, True, False)
    return carry
  lax.fori_loop(1, rounds, round_body, 0)

  for c in range(rounds * NB, nch):
    phase(c, c % NB, False, False)
  for b in range(NB):
    wait_scatter(b)


def _scale_rows(rows_v, w_v, c, groups):
  def group(g, carry):
    wvec = w_v[pl.ds(c * CH + 16 * g, 16)]
    for k in range(16):
      wsp = wvec.at[jnp.full((16,), k, jnp.int32)].get(
          mode="promise_in_bounds")
      e = 16 * g + k
      for j in range(groups):
        rows_v[e, pl.ds(16 * j, 16)] = rows_v[e, pl.ds(16 * j, 16)] * wsp
    return carry
  lax.fori_loop(0, CH // 16, group, 0)


def _zero_acc(zero_v, acc, sid, groups):
  def zrow(r, carry):
    for j in range(groups):
      zero_v[r, pl.ds(16 * j, 16)] = jnp.zeros((16,), jnp.float32)
    return carry
  lax.fori_loop(0, ZR, zrow, 0)
  for k in range(RPT // ZR):
    pltpu.sync_copy(zero_v, acc.at[pl.ds(sid * RPT + k * ZR, ZR)])
  rem = RPT % ZR
  if rem:
    pltpu.sync_copy(zero_v.at[pl.ds(0, rem)],
                    acc.at[pl.ds(sid * RPT + (RPT // ZR) * ZR, rem)])


def _run_edges(x_hbm, src_v, dst_v, w_v, rows, gsems, ssems, acc, nch, groups):
  NB = len(rows)

  def wait_gather(b, c):
    pltpu.make_async_copy(x_hbm.at[src_v.at[c]], rows[b], gsems[b]).wait()

  def wait_scatter(b):
    pltpu.make_async_copy(rows[b], acc.at[dst_v.at[0]], ssems[b]).wait()

  def phase(c, b, prefetch):
    wait_gather(b, c)
    _scale_rows(rows[b], w_v, c, groups)
    pltpu.async_copy(rows[b], acc.at[dst_v.at[c]], ssems[b], add=True)
    if prefetch:
      cp = c + (NB - 1)
      bp = (b + NB - 1) % NB

      @pl.when(cp < nch)
      def _():
        @pl.when(cp >= NB)
        def _():
          wait_scatter(bp)
        pltpu.async_copy(x_hbm.at[src_v.at[cp]], rows[bp], gsems[bp])

  for b in range(NB - 1):
    pltpu.async_copy(x_hbm.at[src_v.at[b]], rows[b], gsems[b])

  rounds = nch // NB

  def round_body(r, carry):
    for k in range(NB):
      phase(r * NB + k, k, True)
    return carry
  lax.fori_loop(0, rounds, round_body, 0)

  for c in range(rounds * NB, nch):
    phase(c, c % NB, False)
  for b in range(NB):
    wait_scatter(b)


def _sc_scratch(nch, ept, width, nb):
  return [
      pltpu.VMEM((nch, CH), jnp.int32),
      pltpu.VMEM((nch, CH), jnp.int32),
      pltpu.VMEM((ept,), jnp.float32),
      [pltpu.VMEM((CH, width), jnp.float32) for _ in range(nb)],
      pltpu.VMEM_SHARED((N_PAD, width), jnp.float32),
      [pltpu.SemaphoreType.DMA for _ in range(nb)],
      [pltpu.SemaphoreType.DMA for _ in range(nb)],
  ]


def _make_spmm_fsplit(width):
  mesh = plsc.VectorSubcoreMesh(core_axis_name="c", subcore_axis_name="s")
  groups = width // 16

  @functools.partial(
      pl.kernel,
      out_type=jax.ShapeDtypeStruct((NC, N_PAD, width), jnp.float32),
      mesh=mesh,
      compiler_params=pltpu.CompilerParams(
          use_tc_tiling_on_sc=False, needs_layout_passes=False),
      scratch_types=[
          pltpu.VMEM((NCH_F, CH), jnp.int32),
          pltpu.VMEM((NCH_F, CH), jnp.int32),
          pltpu.VMEM((EPT_F,), jnp.float32),
          [pltpu.VMEM((CH, width), jnp.bfloat16) for _ in range(NB_F)],
          [pltpu.VMEM((CH, width), jnp.float32) for _ in range(NB_F)],
          pltpu.VMEM_SHARED((N_PAD, width), jnp.float32),
          [pltpu.SemaphoreType.DMA for _ in range(NB_F)],
          [pltpu.SemaphoreType.DMA for _ in range(NB_F)],
      ],
  )
  def spmm(src_hbm, dst_hbm, w_hbm, x0_hbm, x1_hbm, out_hbm,
           src_v, dst_v, w_v, rows_b, scat, acc, gsems, ssems):
    cid = lax.axis_index("c")
    sid = lax.axis_index("s")

    pltpu.sync_copy(src_hbm.at[sid], src_v)
    pltpu.sync_copy(dst_hbm.at[sid], dst_v)
    pltpu.sync_copy(w_hbm.at[sid], w_v)
    _zero_acc(scat[0], acc, sid, groups)
    plsc.subcore_barrier()

    for half, x_hbm in ((0, x0_hbm), (1, x1_hbm)):
      @pl.when(cid == half)
      def _():
        _run_edges_bf16(x_hbm, src_v, dst_v, w_v, rows_b, scat,
                        gsems, ssems, acc, NCH_F)

    plsc.subcore_barrier()
    pltpu.sync_copy(acc.at[pl.ds(sid * RPT, RPT)],
                    out_hbm.at[cid].at[pl.ds(sid * RPT, RPT)])

  return spmm


def _make_spmm_esplit(width):
  mesh = plsc.VectorSubcoreMesh(core_axis_name="c", subcore_axis_name="s")
  groups = width // 16

  @functools.partial(
      pl.kernel,
      out_type=jax.ShapeDtypeStruct((NC, N_PAD, width), jnp.float32),
      mesh=mesh,
      compiler_params=pltpu.CompilerParams(
          use_tc_tiling_on_sc=False, needs_layout_passes=False),
      scratch_types=_sc_scratch(NCH_E, EPT_E, width, NB_E),
  )
  def spmm(src_hbm, dst_hbm, w_hbm, x_hbm, out_hbm,
           src_v, dst_v, w_v, rows, acc, gsems, ssems):
    cid = lax.axis_index("c")
    sid = lax.axis_index("s")
    wid = sid * NC + cid

    pltpu.sync_copy(src_hbm.at[wid], src_v)
    pltpu.sync_copy(dst_hbm.at[wid], dst_v)
    pltpu.sync_copy(w_hbm.at[wid], w_v)
    _zero_acc(rows[0], acc, sid, groups)
    plsc.subcore_barrier()

    _run_edges(x_hbm, src_v, dst_v, w_v, rows, gsems, ssems, acc,
               NCH_E, groups)

    plsc.subcore_barrier()
    pltpu.sync_copy(acc.at[pl.ds(sid * RPT, RPT)],
                    out_hbm.at[cid].at[pl.ds(sid * RPT, RPT)])

  return spmm


_spmm_f64 = _make_spmm_fsplit(64)
_spmm_e16 = _make_spmm_esplit(16)


_RB = 2000


def _dense_body(pl_ref, ph_ref, w_ref, b_ref, o0_ref, o1_ref):
  h = jnp.concatenate([pl_ref[0], ph_ref[0]], axis=1)
  y = jnp.dot(h, w_ref[...], preferred_element_type=jnp.float32) + b_ref[...]
  y = jnp.maximum(y, 0.0)
  o0_ref[...] = y[:, :64].astype(jnp.bfloat16)
  o1_ref[...] = y[:, 64:].astype(jnp.bfloat16)


def _dense_relu(p, w, b):
  return pl.pallas_call(
      _dense_body,
      grid=(N // _RB,),
      in_specs=[
          pl.BlockSpec((1, _RB, 64), lambda i: (0, i, 0)),
          pl.BlockSpec((1, _RB, 64), lambda i: (1, i, 0)),
          pl.BlockSpec((HID, HID), lambda i: (0, 0)),
          pl.BlockSpec((1, HID), lambda i: (0, 0)),
      ],
      out_specs=[
          pl.BlockSpec((_RB, 64), lambda i: (i, 0)),
          pl.BlockSpec((_RB, 64), lambda i: (i, 0)),
      ],
      out_shape=[
          jax.ShapeDtypeStruct((N, 64), jnp.bfloat16),
          jax.ShapeDtypeStruct((N, 64), jnp.bfloat16),
      ],
  )(p, p, w, b)


def _dense2_body(pl_ref, ph_ref, w2_ref, b2_ref, w3_ref, vis_ref, t_ref):
  h = jnp.concatenate([pl_ref[0], ph_ref[0]], axis=1)
  h2 = jnp.maximum(
      jnp.dot(h, w2_ref[...], preferred_element_type=jnp.float32)
      + b2_ref[...], 0.0)
  vis_ref[...] = h2
  t_ref[...] = jnp.dot(h2, w3_ref[...], preferred_element_type=jnp.float32)


def _dense2(p, w2, b2, w3):
  return pl.pallas_call(
      _dense2_body,
      grid=(N // _RB,),
      in_specs=[
          pl.BlockSpec((1, _RB, 64), lambda i: (0, i, 0)),
          pl.BlockSpec((1, _RB, 64), lambda i: (1, i, 0)),
          pl.BlockSpec((HID, HID), lambda i: (0, 0)),
          pl.BlockSpec((1, HID), lambda i: (0, 0)),
          pl.BlockSpec((HID, NCLASS), lambda i: (0, 0)),
      ],
      out_specs=[
          pl.BlockSpec((_RB, HID), lambda i: (i, 0)),
          pl.BlockSpec((_RB, NCLASS), lambda i: (i, 0)),
      ],
      out_shape=[
          jax.ShapeDtypeStruct((N, HID), jnp.float32),
          jax.ShapeDtypeStruct((N, NCLASS), jnp.float32),
      ],
  )(p, p, w2, b2, w3)


def _final_body(p0_ref, p1_ref, b_ref, o_ref):
  logits = p0_ref[0] + p1_ref[0] + b_ref[...]
  m = jnp.max(logits, axis=1, keepdims=True)
  s = logits - m
  lse = jnp.log(jnp.sum(jnp.exp(s), axis=1, keepdims=True))
  o_ref[...] = s - lse


def _final(p, b3):
  return pl.pallas_call(
      _final_body,
      grid=(N // _RB,),
      in_specs=[
          pl.BlockSpec((1, _RB, NCLASS), lambda i: (0, i, 0)),
          pl.BlockSpec((1, _RB, NCLASS), lambda i: (1, i, 0)),
          pl.BlockSpec((1, NCLASS), lambda i: (0, 0)),
      ],
      out_specs=pl.BlockSpec((_RB, NCLASS), lambda i: (i, 0)),
      out_shape=jax.ShapeDtypeStruct((N, NCLASS), jnp.float32),
  )(p, p, b3)


def _pad_edges(a, total):
  return jnp.pad(a, (0, total - E))


def kernel(x, edge_index, edge_weight, W1, b1, W2, b2, W3, b3):
  src32 = edge_index[1].astype(jnp.int32)
  dst32 = edge_index[0].astype(jnp.int32)
  src_f = _pad_edges(src32, NS * EPT_F).reshape(NS, NCH_F, CH)
  dst_f = _pad_edges(dst32, NS * EPT_F).reshape(NS, NCH_F, CH)
  w_f = _pad_edges(edge_weight, NS * EPT_F).reshape(NS, EPT_F)
  src_e = _pad_edges(src32, NC * NS * EPT_E).reshape(NC * NS, NCH_E, CH)
  dst_e = _pad_edges(dst32, NC * NS * EPT_E).reshape(NC * NS, NCH_E, CH)
  w_e = _pad_edges(edge_weight, NC * NS * EPT_E).reshape(NC * NS, EPT_E)

  xb = x.astype(jnp.bfloat16)
  p0 = _spmm_f64(src_f, dst_f, w_f, xb[:, :64], xb[:, 64:])
  h1b0, h1b1 = _dense_relu(p0, W1[_PERM128], b1.reshape(1, HID))
  p1 = _spmm_f64(src_f, dst_f, w_f, h1b0, h1b1)
  vis, t2 = _dense2(p1, W2[_PERM128], b2.reshape(1, HID), W3)
  p2 = _spmm_e16(src_e, dst_e, w_e, t2)
  logp = _final(p2, b3.reshape(1, NCLASS))
  return (logp, vis)
